# baseline (speedup 1.0000x reference)
import jax
import jax.numpy as jnp
from jax import lax
from jax.experimental import pallas as pl
from jax.experimental.pallas import tpu as pltpu
from jax.experimental.pallas import tpu_sc as plsc

P = 10.0
A = 10.0
N_CLAUSES = 100000
N_VARS = 100000
E = 1600000

NC, NS, LANES = 2, 16, 16
PER_TILE = E // NS
CHUNK = 512
NBUF = 4
ROWS = CHUNK // 128
NFULL = PER_TILE // CHUNK
TAIL_START = PER_TILE - CHUNK
TAIL_SLOT = NFULL % NBUF
DUP_EDGES = NFULL * CHUNK - TAIL_START
ACC_N = 100096
SLICE = ACC_N // NS
BOUNCE = SLICE
DUMMY = N_CLAUSES


def _sc_body(*refs):
    (xv_hbm, pos_hbm, neg_hbm, out_hbm, xv_v), rest = refs[:5], refs[5:]
    var_p, rest = rest[:NBUF], rest[NBUF:]
    cls_p, rest = rest[:NBUF], rest[NBUF:]
    cls3_p, rest = rest[:NBUF], rest[NBUF:]
    num3_p, rest = rest[:NBUF], rest[NBUF:]
    den3_p, rest = rest[:NBUF], rest[NBUF:]
    (zbuf, acc_num, acc_den), rest = rest[:3], rest[3:]
    sem_i, sem_s = rest[:NBUF], rest[NBUF:]
    c = lax.axis_index("c")
    s = lax.axis_index("s")

    pltpu.sync_copy(xv_hbm, xv_v)

    def _zero(i, carry):
        zbuf[pl.ds(i * LANES, LANES)] = jnp.zeros((LANES,), jnp.float32)
        return carry
    lax.fori_loop(0, BOUNCE // LANES, _zero, 0)
    off = pl.multiple_of(s * SLICE, 8)
    pltpu.sync_copy(zbuf, acc_num.at[pl.ds(off, SLICE)])
    pltpu.sync_copy(zbuf, acc_den.at[pl.ds(off, SLICE)])
    plsc.subcore_barrier()

    sign = jnp.where(c > 0, -1.0, 1.0).astype(jnp.float32)
    offs = jnp.where(c > 0, 1.0, 0.0).astype(jnp.float32)

    base = s * PER_TILE

    def fire_in(start, b):
        start = pl.multiple_of(start, 8)

        @pl.when(c == 0)
        def _():
            pltpu.async_copy(pos_hbm.at[pl.ds(E + start, CHUNK)], var_p[b],
                             sem_i[b])
            pltpu.async_copy(pos_hbm.at[pl.ds(start, CHUNK)], cls_p[b],
                             sem_i[b])

        @pl.when(c > 0)
        def _():
            pltpu.async_copy(neg_hbm.at[pl.ds(E + start, CHUNK)], var_p[b],
                             sem_i[b])
            pltpu.async_copy(neg_hbm.at[pl.ds(start, CHUNK)], cls_p[b],
                             sem_i[b])

    def wait_in(b):
        pltpu.make_async_copy(pos_hbm.at[pl.ds(0, CHUNK)], var_p[b],
                              sem_i[b]).wait()
        pltpu.make_async_copy(pos_hbm.at[pl.ds(0, CHUNK)], cls_p[b],
                              sem_i[b]).wait()

    def compute(b, tail):
        for i in range(CHUNK // LANES):
            o16 = i * LANES
            r, k_ = i // 8, (i % 8) * LANES
            iv = var_p[b][pl.ds(o16, LANES)]
            xg = plsc.load_gather(xv_v, [iv])
            x = offs + sign * xg
            e = jnp.exp(P * x)
            num3_p[b][r, pl.ds(k_, LANES)] = x * e
            den3_p[b][r, pl.ds(k_, LANES)] = e
            if tail and o16 < DUP_EDGES:
                ci = jnp.full((LANES,), DUMMY, jnp.int32)
            else:
                ci = cls_p[b][pl.ds(o16, LANES)]
            cls3_p[b][r, pl.ds(k_, LANES)] = ci

    def fire_sc(b):
        for r in range(ROWS):
            pltpu.async_copy(num3_p[b].at[r], acc_num.at[cls3_p[b].at[r]],
                             sem_s[b], add=True)
            pltpu.async_copy(den3_p[b].at[r], acc_den.at[cls3_p[b].at[r]],
                             sem_s[b], add=True)

    def drain_sc(b):
        for r in range(ROWS):
            pltpu.make_async_copy(num3_p[b].at[r],
                                  acc_num.at[cls3_p[b].at[r]],
                                  sem_s[b]).wait()
            pltpu.make_async_copy(den3_p[b].at[r],
                                  acc_den.at[cls3_p[b].at[r]],
                                  sem_s[b]).wait()

    def unit_start(u):
        return jnp.where(u == NFULL, base + TAIL_START, base + u * CHUNK)

    for sl in range(NBUF):
        fire_in(unit_start(sl), sl)

    def unit_body(k, carry):
        def process(b):
            wait_in(b)

            @pl.when(k >= NBUF)
            def _():
                drain_sc(b)
            compute(b, False)
            fire_sc(b)

            @pl.when(k + NBUF <= NFULL)
            def _():
                fire_in(unit_start(k + NBUF), b)

        for sl in range(NBUF):
            @pl.when(k % NBUF == sl)
            def _(sl=sl):
                process(sl)
        return carry
    lax.fori_loop(0, NFULL, unit_body, 0)

    wait_in(TAIL_SLOT)
    drain_sc(TAIL_SLOT)
    compute(TAIL_SLOT, True)
    fire_sc(TAIL_SLOT)

    for sl in range(NBUF):
        drain_sc(sl)

    plsc.subcore_barrier()
    onum = pl.multiple_of(c * (2 * ACC_N) + off, 8)
    oden = pl.multiple_of(c * (2 * ACC_N) + ACC_N + off, 8)
    pltpu.sync_copy(acc_num.at[pl.ds(off, SLICE)], zbuf)
    pltpu.sync_copy(zbuf, out_hbm.at[pl.ds(onum, SLICE)])
    pltpu.sync_copy(acc_den.at[pl.ds(off, SLICE)], zbuf)
    pltpu.sync_copy(zbuf, out_hbm.at[pl.ds(oden, SLICE)])


@jax.jit
def _sc_accumulate(xv, pos_flat, neg_flat):
    mesh = plsc.VectorSubcoreMesh(core_axis_name="c", subcore_axis_name="s",
                                  num_cores=NC)
    f = pl.kernel(
        _sc_body,
        out_type=jax.ShapeDtypeStruct((NC * 2 * ACC_N,), jnp.float32),
        mesh=mesh,
        compiler_params=pltpu.CompilerParams(needs_layout_passes=False),
        scratch_types=[
            pltpu.VMEM((N_VARS,), jnp.float32),
            *([pltpu.VMEM((CHUNK,), jnp.int32)] * (2 * NBUF)),
            *([pltpu.VMEM((ROWS, 128), jnp.int32)] * NBUF),
            *([pltpu.VMEM((ROWS, 128), jnp.float32)] * (2 * NBUF)),
            pltpu.VMEM((BOUNCE,), jnp.float32),
            pltpu.VMEM_SHARED((ACC_N,), jnp.float32),
            pltpu.VMEM_SHARED((ACC_N,), jnp.float32),
            *([pltpu.SemaphoreType.DMA] * (2 * NBUF)),
        ],
    )
    return f(xv, pos_flat, neg_flat)


def _tc_body(p_ref, o_ref):
    x = p_ref[...]
    num = x[0, 0] + x[1, 0]
    den = x[0, 1] + x[1, 1]
    t = A * (0.5 - num / den)
    loss = jnp.log(1.0 + jnp.exp(t))
    row = lax.broadcasted_iota(jnp.int32, (ACC_N // 128, 128), 0)
    col = lax.broadcasted_iota(jnp.int32, (ACC_N // 128, 128), 1)
    valid = (row * 128 + col) < N_CLAUSES
    o_ref[0, 0] = jnp.sum(jnp.where(valid, loss, 0.0))


@jax.jit
def _tc_loss(parts):
    return pl.pallas_call(
        _tc_body,
        out_shape=jax.ShapeDtypeStruct((1, 1), jnp.float32),
        out_specs=pl.BlockSpec(memory_space=pltpu.SMEM),
    )(parts)


def kernel(xv, adj_pos, adj_neg):
    xv1 = xv.reshape(-1).astype(jnp.float32)
    pos_flat = adj_pos.astype(jnp.int32).reshape(-1)
    neg_flat = adj_neg.astype(jnp.int32).reshape(-1)
    parts = _sc_accumulate(xv1, pos_flat, neg_flat)
    parts = parts.reshape(NC, 2, ACC_N // 128, 128)
    return _tc_loss(parts)[0, 0]


# speedup vs baseline: 1.0220x; 1.0220x over previous
import jax
import jax.numpy as jnp
from jax import lax
from jax.experimental import pallas as pl
from jax.experimental.pallas import tpu as pltpu
from jax.experimental.pallas import tpu_sc as plsc

P = 10.0
A = 10.0
N_CLAUSES = 100000
N_VARS = 100000
E = 1600000

NC, NS, LANES = 2, 16, 16
PER_TILE = E // NS
CHUNK = 512
NBUF = 4
ROWS = CHUNK // 128
NFULL = PER_TILE // CHUNK
TAIL_START = PER_TILE - CHUNK
TAIL_SLOT = NFULL % NBUF
DUP_EDGES = NFULL * CHUNK - TAIL_START
ACC_N = 100096
SLICE = ACC_N // NS
BOUNCE = SLICE
DUMMY = N_CLAUSES


def _sc_body(*refs):
    (xv_hbm, pos_hbm, neg_hbm, out_hbm, xv_v), rest = refs[:5], refs[5:]
    var_p, rest = rest[:NBUF], rest[NBUF:]
    cls_p, rest = rest[:NBUF], rest[NBUF:]
    num3_p, rest = rest[:NBUF], rest[NBUF:]
    den3_p, rest = rest[:NBUF], rest[NBUF:]
    (zbuf, acc_num, acc_den), rest = rest[:3], rest[3:]
    sem_i, sem_s = rest[:NBUF], rest[NBUF:]
    c = lax.axis_index("c")
    s = lax.axis_index("s")

    pltpu.sync_copy(xv_hbm, xv_v)

    def _zero(i, carry):
        zbuf[pl.ds(i * LANES, LANES)] = jnp.zeros((LANES,), jnp.float32)
        return carry
    lax.fori_loop(0, BOUNCE // LANES, _zero, 0)
    off = pl.multiple_of(s * SLICE, 8)
    pltpu.sync_copy(zbuf, acc_num.at[pl.ds(off, SLICE)])
    pltpu.sync_copy(zbuf, acc_den.at[pl.ds(off, SLICE)])
    plsc.subcore_barrier()

    sign = jnp.where(c > 0, -1.0, 1.0).astype(jnp.float32)
    offs = jnp.where(c > 0, 1.0, 0.0).astype(jnp.float32)

    base = s * PER_TILE

    def fire_in(start, b):
        start = pl.multiple_of(start, 8)

        @pl.when(c == 0)
        def _():
            pltpu.async_copy(pos_hbm.at[pl.ds(E + start, CHUNK)], var_p[b],
                             sem_i[b])
            pltpu.async_copy(pos_hbm.at[pl.ds(start, CHUNK)], cls_p[b],
                             sem_i[b])

        @pl.when(c > 0)
        def _():
            pltpu.async_copy(neg_hbm.at[pl.ds(E + start, CHUNK)], var_p[b],
                             sem_i[b])
            pltpu.async_copy(neg_hbm.at[pl.ds(start, CHUNK)], cls_p[b],
                             sem_i[b])

    def wait_in(b):
        pltpu.make_async_copy(pos_hbm.at[pl.ds(0, CHUNK)], var_p[b],
                              sem_i[b]).wait()
        pltpu.make_async_copy(pos_hbm.at[pl.ds(0, CHUNK)], cls_p[b],
                              sem_i[b]).wait()

    def compute(b, tail):
        for i in range(CHUNK // LANES):
            o16 = i * LANES
            r, k_ = i // 8, (i % 8) * LANES
            iv = var_p[b][pl.ds(o16, LANES)]
            xg = plsc.load_gather(xv_v, [iv])
            x = offs + sign * xg
            e = jnp.exp(P * x)
            num3_p[b][r, pl.ds(k_, LANES)] = x * e
            den3_p[b][r, pl.ds(k_, LANES)] = e
            if tail and o16 < DUP_EDGES:
                cls_p[b][pl.ds(o16, LANES)] = jnp.full((LANES,), DUMMY,
                                                       jnp.int32)

    def fire_sc(b):
        for r in range(ROWS):
            ix = cls_p[b].at[pl.ds(r * 128, 128)]
            pltpu.async_copy(num3_p[b].at[r], acc_num.at[ix],
                             sem_s[b], add=True)
            pltpu.async_copy(den3_p[b].at[r], acc_den.at[ix],
                             sem_s[b], add=True)

    def drain_sc(b):
        for r in range(ROWS):
            ix = cls_p[b].at[pl.ds(r * 128, 128)]
            pltpu.make_async_copy(num3_p[b].at[r], acc_num.at[ix],
                                  sem_s[b]).wait()
            pltpu.make_async_copy(den3_p[b].at[r], acc_den.at[ix],
                                  sem_s[b]).wait()

    def unit_start(u):
        return jnp.where(u == NFULL, base + TAIL_START, base + u * CHUNK)

    for sl in range(NBUF):
        fire_in(unit_start(sl), sl)

    def unit_body(k, carry):
        def process(b):
            wait_in(b)

            @pl.when(k >= NBUF)
            def _():
                drain_sc(b)
            compute(b, False)
            fire_sc(b)

            @pl.when(k + NBUF <= NFULL)
            def _():
                fire_in(unit_start(k + NBUF), b)

        for sl in range(NBUF):
            @pl.when(k % NBUF == sl)
            def _(sl=sl):
                process(sl)
        return carry
    lax.fori_loop(0, NFULL, unit_body, 0)

    wait_in(TAIL_SLOT)
    drain_sc(TAIL_SLOT)
    compute(TAIL_SLOT, True)
    fire_sc(TAIL_SLOT)

    for sl in range(NBUF):
        drain_sc(sl)

    plsc.subcore_barrier()
    onum = pl.multiple_of(c * (2 * ACC_N) + off, 8)
    oden = pl.multiple_of(c * (2 * ACC_N) + ACC_N + off, 8)
    pltpu.sync_copy(acc_num.at[pl.ds(off, SLICE)], zbuf)
    pltpu.sync_copy(zbuf, out_hbm.at[pl.ds(onum, SLICE)])
    pltpu.sync_copy(acc_den.at[pl.ds(off, SLICE)], zbuf)
    pltpu.sync_copy(zbuf, out_hbm.at[pl.ds(oden, SLICE)])


@jax.jit
def _sc_accumulate(xv, pos_flat, neg_flat):
    mesh = plsc.VectorSubcoreMesh(core_axis_name="c", subcore_axis_name="s",
                                  num_cores=NC)
    f = pl.kernel(
        _sc_body,
        out_type=jax.ShapeDtypeStruct((NC * 2 * ACC_N,), jnp.float32),
        mesh=mesh,
        compiler_params=pltpu.CompilerParams(needs_layout_passes=False),
        scratch_types=[
            pltpu.VMEM((N_VARS,), jnp.float32),
            *([pltpu.VMEM((CHUNK,), jnp.int32)] * (2 * NBUF)),
            *([pltpu.VMEM((ROWS, 128), jnp.float32)] * (2 * NBUF)),
            pltpu.VMEM((BOUNCE,), jnp.float32),
            pltpu.VMEM_SHARED((ACC_N,), jnp.float32),
            pltpu.VMEM_SHARED((ACC_N,), jnp.float32),
            *([pltpu.SemaphoreType.DMA] * (2 * NBUF)),
        ],
    )
    return f(xv, pos_flat, neg_flat)


def _tc_body(p_ref, o_ref):
    x = p_ref[...]
    num = x[0, 0] + x[1, 0]
    den = x[0, 1] + x[1, 1]
    t = A * (0.5 - num / den)
    loss = jnp.log(1.0 + jnp.exp(t))
    row = lax.broadcasted_iota(jnp.int32, (ACC_N // 128, 128), 0)
    col = lax.broadcasted_iota(jnp.int32, (ACC_N // 128, 128), 1)
    valid = (row * 128 + col) < N_CLAUSES
    o_ref[0, 0] = jnp.sum(jnp.where(valid, loss, 0.0))


@jax.jit
def _tc_loss(parts):
    return pl.pallas_call(
        _tc_body,
        out_shape=jax.ShapeDtypeStruct((1, 1), jnp.float32),
        out_specs=pl.BlockSpec(memory_space=pltpu.SMEM),
    )(parts)


def kernel(xv, adj_pos, adj_neg):
    xv1 = xv.reshape(-1).astype(jnp.float32)
    pos_flat = adj_pos.astype(jnp.int32).reshape(-1)
    neg_flat = adj_neg.astype(jnp.int32).reshape(-1)
    parts = _sc_accumulate(xv1, pos_flat, neg_flat)
    parts = parts.reshape(NC, 2, ACC_N // 128, 128)
    return _tc_loss(parts)[0, 0]
